# baseline (speedup 1.0000x reference)
import functools

import jax
import jax.numpy as jnp
from jax import lax
from jax.experimental import pallas as pl
from jax.experimental.pallas import tpu as pltpu
from jax.experimental.pallas import tpu_sc as plsc

B = 16384
D = 32
F = 26
NB = 6


@functools.lru_cache(maxsize=1)
def _build_sc_embed():
    info = plsc.get_sparse_core_info()
    NC, NS = info.num_cores, info.num_subcores
    NW = NC * NS
    BPW = B // NW

    mesh = plsc.VectorSubcoreMesh(core_axis_name="c", subcore_axis_name="s")

    @functools.partial(
        pl.kernel,
        out_type=jax.ShapeDtypeStruct((B, F * D), jnp.bfloat16),
        mesh=mesh,
        compiler_params=pltpu.CompilerParams(use_tc_tiling_on_sc=False),
        scratch_types=[
            pltpu.VMEM((F * BPW,), jnp.int32),
            pltpu.VMEM((NB, BPW, D), jnp.bfloat16),
            pltpu.SemaphoreType.DMA,
            [pltpu.SemaphoreType.DMA] * NB,
            [pltpu.SemaphoreType.DMA] * NB,
        ],
    )
    def sc_embed(*refs):
        feats = refs[0:F]
        tables = refs[F:2 * F]
        out = refs[2 * F]
        idx_v, rows_v, isem, gsems, osems = refs[2 * F + 1:]

        wid = lax.axis_index("s") * NC + lax.axis_index("c")
        base = wid * BPW

        idx_h = [
            pltpu.async_copy(
                feats[f].at[pl.ds(base, BPW)],
                idx_v.at[pl.ds(f * BPW, BPW)], isem)
            for f in range(F)
        ]
        for h in idx_h:
            h.wait()

        gh = [None] * F
        out_h = [None] * F

        def fire_field(f):
            buf = f % NB
            if f >= NB:
                out_h[f - NB].wait()
            gh[f] = pltpu.async_copy(
                tables[f].at[idx_v.at[pl.ds(f * BPW, BPW)]],
                rows_v.at[buf], gsems[buf])

        def retire_field(f):
            buf = f % NB
            gh[f].wait()
            out_h[f] = pltpu.async_copy(
                rows_v.at[buf],
                out.at[pl.ds(base, BPW), pl.ds(f * D, D)], osems[buf])

        LAG = NB - 1
        for f in range(F):
            fire_field(f)
            if f >= LAG:
                retire_field(f - LAG)
        for f in range(F - LAG, F):
            retire_field(f)
        for f in range(F - NB, F):
            out_h[f].wait()

    return sc_embed


def kernel(feat_0, feat_1, feat_2, feat_3, feat_4, feat_5, feat_6, feat_7,
           feat_8, feat_9, feat_10, feat_11, feat_12, feat_13, feat_14,
           feat_15, feat_16, feat_17, feat_18, feat_19, feat_20, feat_21,
           feat_22, feat_23, feat_24, feat_25,
           W_0, W_1, W_2, W_3, W_4, W_5, W_6, W_7,
           W_8, W_9, W_10, W_11, W_12, W_13, W_14, W_15,
           W_16, W_17, W_18, W_19, W_20, W_21, W_22, W_23,
           W_24, W_25):
    feats = [feat_0, feat_1, feat_2, feat_3, feat_4, feat_5, feat_6, feat_7,
             feat_8, feat_9, feat_10, feat_11, feat_12, feat_13, feat_14,
             feat_15, feat_16, feat_17, feat_18, feat_19, feat_20, feat_21,
             feat_22, feat_23, feat_24, feat_25]
    tables = [W_0, W_1, W_2, W_3, W_4, W_5, W_6, W_7,
              W_8, W_9, W_10, W_11, W_12, W_13, W_14, W_15,
              W_16, W_17, W_18, W_19, W_20, W_21, W_22, W_23,
              W_24, W_25]
    tables = [w.astype(jnp.bfloat16) for w in tables]
    out = _build_sc_embed()(*feats, *tables)
    return out.astype(jnp.float32)


# speedup vs baseline: 1.4813x; 1.4813x over previous
import functools

import jax
import jax.numpy as jnp
from jax import lax
from jax.experimental import pallas as pl
from jax.experimental.pallas import tpu as pltpu
from jax.experimental.pallas import tpu_sc as plsc

B = 16384
D = 32
F = 26
RC = 128
GROUPS = [(0, 1, 2, 3), (4, 5, 6, 7), (8, 9, 10, 11), (12, 13, 14, 15),
          (16, 17, 18, 19), (20, 21, 22, 23), (24, 25)]
LANES = 16


@functools.lru_cache(maxsize=1)
def _build_sc_embed():
    info = plsc.get_sparse_core_info()
    NC, NS = info.num_cores, info.num_subcores
    NW = NC * NS
    BPW = B // NW
    NCHK = BPW // RC

    mesh = plsc.VectorSubcoreMesh(core_axis_name="c", subcore_axis_name="s")

    units = [(g, c) for g in range(len(GROUPS)) for c in range(NCHK)]

    @functools.partial(
        pl.kernel,
        out_type=jax.ShapeDtypeStruct((B, F * D), jnp.float32),
        mesh=mesh,
        compiler_params=pltpu.CompilerParams(use_tc_tiling_on_sc=False),
        scratch_types=[
            pltpu.VMEM((F * BPW,), jnp.int32),
            pltpu.VMEM((2, 4, RC, D), jnp.float32),
            pltpu.VMEM((2, RC, 4 * D), jnp.float32),
            pltpu.SemaphoreType.DMA,
            [pltpu.SemaphoreType.DMA] * 2,
            [pltpu.SemaphoreType.DMA] * 2,
        ],
    )
    def sc_embed(*refs):
        feats = refs[0:F]
        tables = refs[F:2 * F]
        out = refs[2 * F]
        idx_v, land_v, stage_v, isem, gsems, osems = refs[2 * F + 1:]

        wid = lax.axis_index("s") * NC + lax.axis_index("c")
        base = wid * BPW

        idx_h = [
            pltpu.async_copy(
                feats[f].at[pl.ds(base, BPW)],
                idx_v.at[pl.ds(f * BPW, BPW)], isem)
            for f in range(F)
        ]
        for h in idx_h:
            h.wait()

        NU = len(units)
        gh = [None] * NU
        out_h = [None] * NU

        def fire_unit(u):
            g, c = units[u]
            buf = u & 1
            if u >= 2:
                out_h[u - 2].wait()
            gh[u] = [
                pltpu.async_copy(
                    tables[f].at[idx_v.at[pl.ds(f * BPW + c * RC, RC)]],
                    land_v.at[buf, k], gsems[buf])
                for k, f in enumerate(GROUPS[g])
            ]

        def retire_unit(u):
            g, c = units[u]
            buf = u & 1
            fields = GROUPS[g]
            for h in gh[u]:
                h.wait()

            def body(r, _):
                for k in range(len(fields)):
                    stage_v[buf, r, pl.ds(k * D, LANES)] = (
                        land_v[buf, k, r, pl.ds(0, LANES)])
                    stage_v[buf, r, pl.ds(k * D + LANES, LANES)] = (
                        land_v[buf, k, r, pl.ds(LANES, LANES)])
                return ()

            lax.fori_loop(0, RC, body, (), unroll=False)
            cw = len(fields) * D
            out_h[u] = pltpu.async_copy(
                stage_v.at[buf, pl.ds(0, RC), pl.ds(0, cw)],
                out.at[pl.ds(base + c * RC, RC),
                       pl.ds(GROUPS[g][0] * D, cw)],
                osems[buf])

        fire_unit(0)
        for u in range(1, NU):
            fire_unit(u)
            retire_unit(u - 1)
        retire_unit(NU - 1)
        out_h[NU - 2].wait()
        out_h[NU - 1].wait()

    return sc_embed


def kernel(feat_0, feat_1, feat_2, feat_3, feat_4, feat_5, feat_6, feat_7,
           feat_8, feat_9, feat_10, feat_11, feat_12, feat_13, feat_14,
           feat_15, feat_16, feat_17, feat_18, feat_19, feat_20, feat_21,
           feat_22, feat_23, feat_24, feat_25,
           W_0, W_1, W_2, W_3, W_4, W_5, W_6, W_7,
           W_8, W_9, W_10, W_11, W_12, W_13, W_14, W_15,
           W_16, W_17, W_18, W_19, W_20, W_21, W_22, W_23,
           W_24, W_25):
    feats = [feat_0, feat_1, feat_2, feat_3, feat_4, feat_5, feat_6, feat_7,
             feat_8, feat_9, feat_10, feat_11, feat_12, feat_13, feat_14,
             feat_15, feat_16, feat_17, feat_18, feat_19, feat_20, feat_21,
             feat_22, feat_23, feat_24, feat_25]
    tables = [W_0, W_1, W_2, W_3, W_4, W_5, W_6, W_7,
              W_8, W_9, W_10, W_11, W_12, W_13, W_14, W_15,
              W_16, W_17, W_18, W_19, W_20, W_21, W_22, W_23,
              W_24, W_25]
    return _build_sc_embed()(*feats, *tables)
